# baseline (speedup 1.0000x reference)
import functools

import jax
import jax.numpy as jnp
from jax import lax
from jax.experimental import pallas as pl
from jax.experimental.pallas import tpu as pltpu
from jax.experimental.pallas import tpu_sc as plsc

_B = 1024
_LT = 200
_LI = 20
_H = 128
_DI = 256
_EPS = 1e-12

_NTOK = _B * _LT
_NC = 2
_NS = 16
_NW = _NC * _NS
_CHUNK = 64
_ROWS_PER_W = _NTOK // _NW
_CH_PER_W = _ROWS_PER_W // _CHUNK
_NBUF = 10
_NGRP = _CH_PER_W // _NBUF
_AH = 6


def _sc_gather(table, idx3d):
    mesh = plsc.VectorSubcoreMesh(core_axis_name="c", subcore_axis_name="s")

    @functools.partial(
        pl.kernel,
        mesh=mesh,
        out_type=jax.ShapeDtypeStruct((_NTOK, _H), jnp.float32),
        scratch_types=[
            pltpu.VMEM((_CH_PER_W, _CHUNK), jnp.int32),
            pltpu.VMEM((_NBUF, _CHUNK, _H), jnp.float32),
        ]
        + [pltpu.SemaphoreType.DMA] * (2 * _NBUF),
    )
    def gather_kernel(table_hbm, idx_hbm, out_hbm, idx_v, rows_v, *sems):
        gsem = sems[:_NBUF]
        ssem = sems[_NBUF:]
        wid = lax.axis_index("s") * _NC + lax.axis_index("c")
        pltpu.sync_copy(idx_hbm.at[wid], idx_v)
        base = wid * _ROWS_PER_W

        def fire_gather(j, b):
            pltpu.async_copy(table_hbm.at[idx_v.at[j]], rows_v.at[b], gsem[b])

        for b in range(_NBUF):
            fire_gather(b, b)

        def grp(g, carry):
            for b in range(_NBUF):
                j = g * _NBUF + b
                pltpu.make_async_copy(
                    table_hbm.at[idx_v.at[0]], rows_v.at[b], gsem[b]
                ).wait()
                pltpu.async_copy(
                    rows_v.at[b],
                    out_hbm.at[pl.ds(base + j * _CHUNK, _CHUNK)],
                    ssem[b],
                )
                bk = (b + _AH) % _NBUF
                k = j + _AH

                @pl.when(k >= _NBUF)
                def _():
                    pltpu.make_async_copy(
                        rows_v.at[bk],
                        out_hbm.at[pl.ds(base, _CHUNK)],
                        ssem[bk],
                    ).wait()

                @pl.when((k >= _NBUF) & (k < _CH_PER_W))
                def _():
                    fire_gather(k, bk)

            return carry

        lax.fori_loop(0, _NGRP, grp, 0)

        for jj in range(_CH_PER_W - (_NBUF - _AH), _CH_PER_W):
            pltpu.make_async_copy(
                rows_v.at[jj % _NBUF], out_hbm.at[pl.ds(base, _CHUNK)],
                ssem[jj % _NBUF],
            ).wait()

    return gather_kernel(table, idx3d)


def _layer_norm_mxu(x, g, b):
    ones = jnp.ones((_H, _H), jnp.float32)
    s1 = jnp.dot(x, ones, preferred_element_type=jnp.float32)
    s2 = jnp.dot(x * x, ones, preferred_element_type=jnp.float32)
    m = s1 * (1.0 / _H)
    v = jnp.maximum(s2 * (1.0 / _H) - m * m, 0.0)
    return (x - m) * (lax.rsqrt(v + _EPS) * g) + b


_BB = 512
_BBI = 256
_TS = 40
_NSEQ_T = _LT // _TS
_LSEQ = _LT + _LI


def _tc_image_body(imf_ref, isid_ref, tte_ref, w_ref, bias_ref,
                   g1_ref, b1_ref, gi_ref, bi_ref, out_ref):
    x = imf_ref[...].reshape(_LI * _BBI, _DI)
    ie = jnp.dot(x, w_ref[...], preferred_element_type=jnp.float32)
    ie = _layer_norm_mxu(ie + bias_ref[0], g1_ref[0], b1_ref[0])
    ie = ie.reshape(_LI, _BBI, _H)
    iseg = isid_ref[...]
    ie = ie + jnp.where(iseg == 0, tte_ref[0, :], tte_ref[1, :])
    ie = _layer_norm_mxu(ie.reshape(_LI * _BBI, _H), gi_ref[0], bi_ref[0])
    out_ref[...] = ie.reshape(_LI, _BBI, _H)


def _tc_image(imgf_t, isid_t, tte, w, bias, g1, b1, gi, bi):
    return pl.pallas_call(
        _tc_image_body,
        grid=(_B // _BBI,),
        in_specs=[
            pl.BlockSpec((_LI, _BBI, _DI), lambda j: (0, j, 0)),
            pl.BlockSpec((_LI, _BBI, 1), lambda j: (0, j, 0)),
            pl.BlockSpec((2, _H), lambda j: (0, 0)),
            pl.BlockSpec((_DI, _H), lambda j: (0, 0)),
            pl.BlockSpec((1, _H), lambda j: (0, 0)),
            pl.BlockSpec((1, _H), lambda j: (0, 0)),
            pl.BlockSpec((1, _H), lambda j: (0, 0)),
            pl.BlockSpec((1, _H), lambda j: (0, 0)),
            pl.BlockSpec((1, _H), lambda j: (0, 0)),
        ],
        out_specs=pl.BlockSpec((_LI, _BBI, _H), lambda j: (_LT // _LI, j, 0)),
        out_shape=jax.ShapeDtypeStruct((_LSEQ, _B, _H), jnp.float32),
    )(imgf_t, isid_t, tte, w, bias, g1, b1, gi, bi)


def _tc_text_body(buf_ref, te_ref, tsid_ref, tte_ref, gt_ref, bt_ref, out_ref):
    del buf_ref
    seg = tsid_ref[...]
    te = te_ref[...] + jnp.where(seg == 0, tte_ref[0, :], tte_ref[1, :])
    te = _layer_norm_mxu(te.reshape(_TS * _BB, _H), gt_ref[0], bt_ref[0])
    out_ref[...] = te.reshape(_TS, _BB, _H)


def _tc_text(buf, te_t, tsid_t, tte, gt, bt):
    return pl.pallas_call(
        _tc_text_body,
        grid=(_B // _BB, _NSEQ_T),
        in_specs=[
            pl.BlockSpec(memory_space=pl.ANY),
            pl.BlockSpec((_TS, _BB, _H), lambda j, i: (i, j, 0)),
            pl.BlockSpec((_TS, _BB, 1), lambda j, i: (i, j, 0)),
            pl.BlockSpec((2, _H), lambda j, i: (0, 0)),
            pl.BlockSpec((1, _H), lambda j, i: (0, 0)),
            pl.BlockSpec((1, _H), lambda j, i: (0, 0)),
        ],
        out_specs=pl.BlockSpec((_TS, _BB, _H), lambda j, i: (i, j, 0)),
        out_shape=jax.ShapeDtypeStruct((_LSEQ, _B, _H), jnp.float32),
        input_output_aliases={0: 0},
    )(buf, te_t, tsid_t, tte, gt, bt)


def kernel(text_input_ids, image_features, text_position_ids, image_position_ids,
           text_segment_ids, image_segment_ids,
           word_emb, token_type_emb, img_W, img_b,
           img_proj_ln_g, img_proj_ln_b, text_ln_g, text_ln_b,
           img_ln_g, img_ln_b):
    del text_position_ids, image_position_ids

    idx3d = text_input_ids.T.reshape(_NW, _CH_PER_W, _CHUNK)
    te_raw = _sc_gather(word_emb, idx3d)
    te_t = te_raw.reshape(_LT, _B, _H)

    row = lambda p: p.reshape(1, _H)
    buf = _tc_image(
        image_features.transpose(1, 0, 2),
        image_segment_ids.astype(jnp.int8).T.reshape(_LI, _B, 1),
        token_type_emb, img_W, row(img_b),
        row(img_proj_ln_g), row(img_proj_ln_b),
        row(img_ln_g), row(img_ln_b),
    )
    out_t = _tc_text(
        buf, te_t,
        text_segment_ids.astype(jnp.int8).T.reshape(_LT, _B, 1),
        token_type_emb, row(text_ln_g), row(text_ln_b),
    )
    return out_t.transpose(1, 0, 2)


# speedup vs baseline: 1.0085x; 1.0085x over previous
import functools

import jax
import jax.numpy as jnp
from jax import lax
from jax.experimental import pallas as pl
from jax.experimental.pallas import tpu as pltpu
from jax.experimental.pallas import tpu_sc as plsc

_B = 1024
_LT = 200
_LI = 20
_H = 128
_DI = 256
_EPS = 1e-12

_NTOK = _B * _LT
_NC = 2
_NS = 16
_NW = _NC * _NS
_CHUNK = 128
_ROWS_PER_W = _NTOK // _NW
_CH_PER_W = _ROWS_PER_W // _CHUNK
_NBUF = 5
_NGRP = _CH_PER_W // _NBUF
_AH = 3


def _sc_gather(table, idx3d):
    mesh = plsc.VectorSubcoreMesh(core_axis_name="c", subcore_axis_name="s")

    @functools.partial(
        pl.kernel,
        mesh=mesh,
        out_type=jax.ShapeDtypeStruct((_NTOK, _H), jnp.float32),
        scratch_types=[
            pltpu.VMEM((_CH_PER_W, _CHUNK), jnp.int32),
            pltpu.VMEM((_NBUF, _CHUNK, _H), jnp.float32),
        ]
        + [pltpu.SemaphoreType.DMA] * (2 * _NBUF),
    )
    def gather_kernel(table_hbm, idx_hbm, out_hbm, idx_v, rows_v, *sems):
        gsem = sems[:_NBUF]
        ssem = sems[_NBUF:]
        wid = lax.axis_index("s") * _NC + lax.axis_index("c")
        pltpu.sync_copy(idx_hbm.at[wid], idx_v)
        base = wid * _ROWS_PER_W

        def fire_gather(j, b):
            pltpu.async_copy(table_hbm.at[idx_v.at[j]], rows_v.at[b], gsem[b])

        for b in range(_NBUF):
            fire_gather(b, b)

        def grp(g, carry):
            for b in range(_NBUF):
                j = g * _NBUF + b
                pltpu.make_async_copy(
                    table_hbm.at[idx_v.at[0]], rows_v.at[b], gsem[b]
                ).wait()
                pltpu.async_copy(
                    rows_v.at[b],
                    out_hbm.at[pl.ds(base + j * _CHUNK, _CHUNK)],
                    ssem[b],
                )
                bk = (b + _AH) % _NBUF
                k = j + _AH

                @pl.when(k >= _NBUF)
                def _():
                    pltpu.make_async_copy(
                        rows_v.at[bk],
                        out_hbm.at[pl.ds(base, _CHUNK)],
                        ssem[bk],
                    ).wait()

                @pl.when((k >= _NBUF) & (k < _CH_PER_W))
                def _():
                    fire_gather(k, bk)

            return carry

        lax.fori_loop(0, _NGRP, grp, 0)

        for jj in range(_CH_PER_W - (_NBUF - _AH), _CH_PER_W):
            pltpu.make_async_copy(
                rows_v.at[jj % _NBUF], out_hbm.at[pl.ds(base, _CHUNK)],
                ssem[jj % _NBUF],
            ).wait()

    return gather_kernel(table, idx3d)


def _layer_norm_mxu(x, g, b):
    ones = jnp.ones((_H, _H), jnp.float32)
    s1 = jnp.dot(x, ones, preferred_element_type=jnp.float32)
    s2 = jnp.dot(x * x, ones, preferred_element_type=jnp.float32)
    m = s1 * (1.0 / _H)
    v = jnp.maximum(s2 * (1.0 / _H) - m * m, 0.0)
    return (x - m) * (lax.rsqrt(v + _EPS) * g) + b


_BB = 512
_BBI = 256
_TS = 40
_NSEQ_T = _LT // _TS
_LSEQ = _LT + _LI


def _tc_image_body(imf_ref, isid_ref, tte_ref, w_ref, bias_ref,
                   g1_ref, b1_ref, gi_ref, bi_ref, out_ref):
    x = imf_ref[...].reshape(_LI * _BBI, _DI)
    ie = jnp.dot(x, w_ref[...], preferred_element_type=jnp.float32)
    ie = _layer_norm_mxu(ie + bias_ref[0], g1_ref[0], b1_ref[0])
    ie = ie.reshape(_LI, _BBI, _H)
    iseg = isid_ref[...]
    ie = ie + jnp.where(iseg == 0, tte_ref[0, :], tte_ref[1, :])
    ie = _layer_norm_mxu(ie.reshape(_LI * _BBI, _H), gi_ref[0], bi_ref[0])
    out_ref[...] = ie.reshape(_LI, _BBI, _H)


def _tc_image(imgf_t, isid_t, tte, w, bias, g1, b1, gi, bi):
    return pl.pallas_call(
        _tc_image_body,
        grid=(_B // _BBI,),
        in_specs=[
            pl.BlockSpec((_LI, _BBI, _DI), lambda j: (0, j, 0)),
            pl.BlockSpec((_LI, _BBI, 1), lambda j: (0, j, 0)),
            pl.BlockSpec((2, _H), lambda j: (0, 0)),
            pl.BlockSpec((_DI, _H), lambda j: (0, 0)),
            pl.BlockSpec((1, _H), lambda j: (0, 0)),
            pl.BlockSpec((1, _H), lambda j: (0, 0)),
            pl.BlockSpec((1, _H), lambda j: (0, 0)),
            pl.BlockSpec((1, _H), lambda j: (0, 0)),
            pl.BlockSpec((1, _H), lambda j: (0, 0)),
        ],
        out_specs=pl.BlockSpec((_LI, _BBI, _H), lambda j: (_LT // _LI, j, 0)),
        out_shape=jax.ShapeDtypeStruct((_LSEQ, _B, _H), jnp.float32),
    )(imgf_t, isid_t, tte, w, bias, g1, b1, gi, bi)


def _tc_text_body(buf_ref, te_ref, tsid_ref, tte_ref, gt_ref, bt_ref, out_ref):
    del buf_ref
    seg = tsid_ref[...]
    te = te_ref[...] + jnp.where(seg == 0, tte_ref[0, :], tte_ref[1, :])
    te = _layer_norm_mxu(te.reshape(_TS * _BB, _H), gt_ref[0], bt_ref[0])
    out_ref[...] = te.reshape(_TS, _BB, _H)


def _tc_text(buf, te_t, tsid_t, tte, gt, bt):
    return pl.pallas_call(
        _tc_text_body,
        grid=(_B // _BB, _NSEQ_T),
        in_specs=[
            pl.BlockSpec(memory_space=pl.ANY),
            pl.BlockSpec((_TS, _BB, _H), lambda j, i: (i, j, 0)),
            pl.BlockSpec((_TS, _BB, 1), lambda j, i: (i, j, 0)),
            pl.BlockSpec((2, _H), lambda j, i: (0, 0)),
            pl.BlockSpec((1, _H), lambda j, i: (0, 0)),
            pl.BlockSpec((1, _H), lambda j, i: (0, 0)),
        ],
        out_specs=pl.BlockSpec((_TS, _BB, _H), lambda j, i: (i, j, 0)),
        out_shape=jax.ShapeDtypeStruct((_LSEQ, _B, _H), jnp.float32),
        input_output_aliases={0: 0},
    )(buf, te_t, tsid_t, tte, gt, bt)


def kernel(text_input_ids, image_features, text_position_ids, image_position_ids,
           text_segment_ids, image_segment_ids,
           word_emb, token_type_emb, img_W, img_b,
           img_proj_ln_g, img_proj_ln_b, text_ln_g, text_ln_b,
           img_ln_g, img_ln_b):
    del text_position_ids, image_position_ids

    idx3d = text_input_ids.T.reshape(_NW, _CH_PER_W, _CHUNK)
    te_raw = _sc_gather(word_emb, idx3d)
    te_t = te_raw.reshape(_LT, _B, _H)

    row = lambda p: p.reshape(1, _H)
    buf = _tc_image(
        image_features.transpose(1, 0, 2),
        image_segment_ids.astype(jnp.int8).T.reshape(_LI, _B, 1),
        token_type_emb, img_W, row(img_b),
        row(img_proj_ln_g), row(img_proj_ln_b),
        row(img_ln_g), row(img_ln_b),
    )
    out_t = _tc_text(
        buf, te_t,
        text_segment_ids.astype(jnp.int8).T.reshape(_LT, _B, 1),
        token_type_emb, row(text_ln_g), row(text_ln_b),
    )
    return out_t.transpose(1, 0, 2)
